# Initial kernel scaffold; baseline (speedup 1.0000x reference)
import jax
import jax.numpy as jnp
from jax.experimental import pallas as pl


def kernel(input_features, modality_indices, embedding_weight):
    raise NotImplementedError("write your pallas kernel here")


# speedup vs baseline: 1.0148x; 1.0148x over previous
import jax
import jax.numpy as jnp
from jax.experimental import pallas as pl
from jax.experimental.pallas import tpu as pltpu

T = 16384
D = 2048
BT = 1024


def _add_kernel(idx_ref, emb_ref, x_ref, o_ref):
    i = idx_ref[0]
    row = emb_ref[pl.ds(i, 1), :]
    o_ref[0] = x_ref[...] + row


def kernel(input_features, modality_indices, embedding_weight):
    grid = (T // BT,)
    out = pl.pallas_call(
        _add_kernel,
        grid_spec=pltpu.PrefetchScalarGridSpec(
            num_scalar_prefetch=1,
            grid=grid,
            in_specs=[
                pl.BlockSpec((4, D), lambda i, idx: (0, 0)),
                pl.BlockSpec((BT, D), lambda i, idx: (i, 0)),
            ],
            out_specs=pl.BlockSpec((1, BT, D), lambda i, idx: (0, i, 0)),
        ),
        out_shape=jax.ShapeDtypeStruct((1, T, D), input_features.dtype),
        compiler_params=pltpu.CompilerParams(
            dimension_semantics=("arbitrary",),
        ),
    )(modality_indices, embedding_weight, input_features)
    return out
